# Initial kernel scaffold; baseline (speedup 1.0000x reference)
import jax
import jax.numpy as jnp
from jax.experimental import pallas as pl


def kernel(x, emb):
    raise NotImplementedError("write your pallas kernel here")


# speedup vs baseline: 14.2091x; 14.2091x over previous
import jax
import jax.numpy as jnp
from jax import lax
from jax.experimental import pallas as pl

_B = 2048


def _body(xt_ref, emb_ref, out_ref):
    xt = xt_ref[...]
    f, b = xt.shape
    v = emb_ref.shape[0] // f
    iota = lax.broadcasted_iota(jnp.int32, (v, b), 0)
    parts = [(xt[i : i + 1] == iota).astype(jnp.float32) for i in range(f)]
    oh_t = jnp.concatenate(parts, axis=0)
    out_ref[...] = lax.dot_general(
        oh_t,
        emb_ref[...],
        ((( 0,), (0,)), ((), ())),
        preferred_element_type=jnp.float32,
    )


def kernel(x, emb):
    n, f = x.shape
    _, v, h = emb.shape
    grid = -(-n // _B)
    n_pad = grid * _B
    xt = jnp.zeros((f, n_pad), jnp.int32).at[:, :n].set(x.T)
    emb_flat = emb.reshape(f * v, h)
    out = pl.pallas_call(
        _body,
        grid=(grid,),
        in_specs=[
            pl.BlockSpec((f, _B), lambda i: (0, i)),
            pl.BlockSpec((f * v, h), lambda i: (0, 0)),
        ],
        out_specs=pl.BlockSpec((_B, h), lambda i: (i, 0)),
        out_shape=jax.ShapeDtypeStruct((n_pad, h), jnp.float32),
    )(xt, emb_flat)
    return out[:n]
